# baseline (speedup 1.0000x reference)
import jax
import jax.numpy as jnp
from jax import lax
from jax.experimental import pallas as pl
from jax.experimental.pallas import tpu as pltpu
from jax.experimental.pallas import tpu_sc as plsc

B = 256
OBS = 1024
H = 2048
DEPTH = 100
NOUT = 64
HEADW = 128

_ROWS = B * DEPTH
_NW = 32
_GW = 16


def _gather_body(stackf_hbm, idx_hbm, top_hbm, idx_v, flat_v, rows_v, sem):
    c = lax.axis_index("c")
    s = lax.axis_index("s")
    wid = s * 2 + c

    @pl.when(wid < _GW)
    def _():
        base = wid * (B // _GW)
        pltpu.sync_copy(idx_hbm.at[pl.ds(base, 16)], idx_v)
        rows = lax.broadcasted_iota(jnp.int32, (16,), 0) + base
        flat_v[...] = rows * DEPTH + idx_v[...]
        pltpu.async_copy(stackf_hbm.at[flat_v], rows_v, sem).wait()
        pltpu.sync_copy(rows_v, top_hbm.at[pl.ds(base, 16)])


def _sc_gather(stack_flat, stack_idx):
    mesh = plsc.VectorSubcoreMesh(core_axis_name="c", subcore_axis_name="s")
    return pl.kernel(
        _gather_body,
        out_type=jax.ShapeDtypeStruct((B, H), jnp.float32),
        mesh=mesh,
        scratch_types=[
            pltpu.VMEM((16,), jnp.int32),
            pltpu.VMEM((16,), jnp.int32),
            pltpu.VMEM((16, H), jnp.float32),
            pltpu.SemaphoreType.DMA,
        ],
    )(stack_flat, stack_idx)


def _copy_body(stackf_hbm, out_hbm, sem):
    c = lax.axis_index("c")
    s = lax.axis_index("s")
    wid = s * 2 + c
    rows_per = _ROWS // _NW
    base = wid * rows_per
    pltpu.async_copy(
        stackf_hbm.at[pl.ds(base, rows_per)],
        out_hbm.at[pl.ds(base, rows_per)],
        sem,
    ).wait()


def _sc_copy(stack_flat):
    mesh = plsc.VectorSubcoreMesh(core_axis_name="c", subcore_axis_name="s")
    return pl.kernel(
        _copy_body,
        out_type=jax.ShapeDtypeStruct((_ROWS, H), jnp.float32),
        mesh=mesh,
        scratch_types=[pltpu.SemaphoreType.DMA],
    )(stack_flat)


def _mlp_body(x_ref, top_ref, w1x_ref, w1t_ref, b1_ref, w2_ref, b2_ref,
              wh_ref, bh_ref, idx_ref, p_ref, logits_ref, value_ref,
              nidx_ref):
    h = jnp.tanh(
        jnp.dot(x_ref[...], w1x_ref[...], preferred_element_type=jnp.float32)
        + jnp.dot(top_ref[...], w1t_ref[...], preferred_element_type=jnp.float32)
        + b1_ref[...]
    )
    p = jnp.tanh(
        jnp.dot(h, w2_ref[...], preferred_element_type=jnp.float32)
        + b2_ref[...]
    )
    p_ref[...] = p
    ph = jnp.dot(p, wh_ref[...], preferred_element_type=jnp.float32) + bh_ref[...]
    logits_ref[...] = ph[:, 3:3 + NOUT]
    value_ref[...] = ph[:, 3 + NOUT:4 + NOUT]
    s0 = ph[:, 0:1]
    s1 = ph[:, 1:2]
    s2 = ph[:, 2:3]
    op = jnp.where(s1 > s0, 1, 0)
    best = jnp.maximum(s0, s1)
    op = jnp.where(s2 > best, 2, op)
    nidx_ref[...] = jnp.maximum(idx_ref[...] + op - 1, 0)


def _tc_mlp(x, top, w1x, w1t, b1, w2, b2, wh, bh, idx2d):
    return pl.pallas_call(
        _mlp_body,
        out_shape=(
            jax.ShapeDtypeStruct((B, H), jnp.float32),
            jax.ShapeDtypeStruct((B, NOUT), jnp.float32),
            jax.ShapeDtypeStruct((B, 1), jnp.float32),
            jax.ShapeDtypeStruct((B, 1), jnp.int32),
        ),
    )(x, top, w1x, w1t, b1, w2, b2, wh, bh, idx2d)


def _scatter_body(ns_in, p_ref, idx_ref, ns_out, sem):
    def start(r, carry):
        d = idx_ref[r] + 1
        pltpu.make_async_copy(p_ref.at[r], ns_out.at[r, d], sem).start()
        return carry

    lax.fori_loop(0, B, start, 0)

    def drain(r, carry):
        d = idx_ref[r] + 1
        pltpu.make_async_copy(p_ref.at[r], ns_out.at[r, d], sem).wait()
        return carry

    lax.fori_loop(0, B, drain, 0)


def _tc_scatter(new_stack, p, stack_idx):
    return pl.pallas_call(
        _scatter_body,
        out_shape=jax.ShapeDtypeStruct((B, DEPTH, H), jnp.float32),
        in_specs=[
            pl.BlockSpec(memory_space=pl.ANY),
            pl.BlockSpec(memory_space=pltpu.VMEM),
            pl.BlockSpec(memory_space=pltpu.SMEM),
        ],
        out_specs=pl.BlockSpec(memory_space=pl.ANY),
        input_output_aliases={0: 0},
        scratch_shapes=[pltpu.SemaphoreType.DMA],
    )(new_stack, p, stack_idx)


def kernel(x, stack, stack_idx, W1, b1, W2, b2, Ws, bs, Wp, bp, Wv, bv):
    stack_flat = stack.reshape(_ROWS, H)

    top = _sc_gather(stack_flat, stack_idx)

    w1x = W1[:OBS]
    w1t = W1[OBS:]
    wh = jnp.zeros((H, HEADW), jnp.float32)
    wh = wh.at[:, 0:3].set(Ws).at[:, 3:3 + NOUT].set(Wp)
    wh = wh.at[:, 3 + NOUT:4 + NOUT].set(Wv)
    bh = jnp.zeros((1, HEADW), jnp.float32)
    bh = bh.at[0, 0:3].set(bs).at[0, 3:3 + NOUT].set(bp)
    bh = bh.at[0, 3 + NOUT].set(bv[0])

    p, logits, value, nidx = _tc_mlp(
        x, top, w1x, w1t, b1.reshape(1, H), W2, b2.reshape(1, H), wh, bh,
        stack_idx.reshape(B, 1),
    )

    new_stack_copy = _sc_copy(stack_flat).reshape(B, DEPTH, H)
    new_stack = _tc_scatter(new_stack_copy, p, stack_idx)

    return (logits, value[:, 0], new_stack, nidx[:, 0])


# speedup vs baseline: 8.0791x; 8.0791x over previous
import jax
import jax.numpy as jnp
from jax import lax
from jax.experimental import pallas as pl
from jax.experimental.pallas import tpu as pltpu
from jax.experimental.pallas import tpu_sc as plsc

B = 256
OBS = 1024
H = 2048
DEPTH = 100
NOUT = 64
HEADW = 128

_ROWS = B * DEPTH
_GW = 16

_BB = 32
_NB = B // _BB
_ND = 8
_FR = _BB * DEPTH // _ND
_CH = H // 4


def _gather_body(stackf_hbm, idx_hbm, top_hbm, idx_v, flat_v, rows_v, sem):
    c = lax.axis_index("c")
    s = lax.axis_index("s")
    wid = s * 2 + c

    @pl.when(wid < _GW)
    def _():
        base = wid * (B // _GW)
        pltpu.sync_copy(idx_hbm.at[pl.ds(base, 16)], idx_v)
        rows = lax.broadcasted_iota(jnp.int32, (16,), 0) + base
        flat_v[...] = rows * DEPTH + idx_v[...]
        pltpu.async_copy(stackf_hbm.at[flat_v], rows_v, sem).wait()
        pltpu.sync_copy(rows_v, top_hbm.at[pl.ds(base, 16)])


def _sc_gather(stack_flat, stack_idx):
    mesh = plsc.VectorSubcoreMesh(core_axis_name="c", subcore_axis_name="s")
    return pl.kernel(
        _gather_body,
        out_type=jax.ShapeDtypeStruct((B, H), jnp.float32),
        mesh=mesh,
        scratch_types=[
            pltpu.VMEM((16,), jnp.int32),
            pltpu.VMEM((16,), jnp.int32),
            pltpu.VMEM((16, H), jnp.float32),
            pltpu.SemaphoreType.DMA,
        ],
    )(stack_flat, stack_idx)


def _mega_body(x_ref, top_ref, w1x_ref, w1t_ref, b1_ref, w2_ref, b2_ref,
               wh_ref, bh_ref, idx_ref, stack_ref, ns_ref, p_ref,
               logits_ref, value_ref, nidx_ref, h_scr):
    ns_ref[...] = stack_ref[...]
    step = pl.program_id(1)

    for j in range(4):
        @pl.when(step == j)
        def _(j=j):
            c0 = j * _CH
            h_scr[:, c0:c0 + _CH] = jnp.tanh(
                jnp.dot(x_ref[...], w1x_ref[:, c0:c0 + _CH],
                        preferred_element_type=jnp.float32)
                + jnp.dot(top_ref[...], w1t_ref[:, c0:c0 + _CH],
                          preferred_element_type=jnp.float32)
                + b1_ref[:, c0:c0 + _CH]
            )

    for j in range(4):
        @pl.when(step == 4 + j)
        def _(j=j):
            c0 = j * _CH
            p_ref[:, c0:c0 + _CH] = jnp.tanh(
                jnp.dot(h_scr[...], w2_ref[:, c0:c0 + _CH],
                        preferred_element_type=jnp.float32)
                + b2_ref[:, c0:c0 + _CH]
            )

    @pl.when(step == _ND - 1)
    def _():
        p = p_ref[...]
        ph = jnp.dot(p, wh_ref[...], preferred_element_type=jnp.float32)
        ph = ph + bh_ref[...]
        logits_ref[...] = ph[:, 3:3 + NOUT]
        value_ref[...] = ph[:, 3 + NOUT:4 + NOUT]
        s0 = ph[:, 0:1]
        s1 = ph[:, 1:2]
        s2 = ph[:, 2:3]
        op = jnp.where(s1 > s0, 1, 0)
        best = jnp.maximum(s0, s1)
        op = jnp.where(s2 > best, 2, op)
        nidx_ref[...] = jnp.maximum(idx_ref[...] + op - 1, 0)


def _tc_mega(x, top, w1x, w1t, b1, w2, b2, wh, bh, idx2d, stack):
    const = lambda ib, id_: (0, 0)
    return pl.pallas_call(
        _mega_body,
        grid=(_NB, _ND),
        in_specs=[
            pl.BlockSpec((_BB, OBS), lambda ib, id_: (ib, 0)),
            pl.BlockSpec((_BB, H), lambda ib, id_: (ib, 0)),
            pl.BlockSpec((OBS, H), const),
            pl.BlockSpec((H, H), const),
            pl.BlockSpec((1, H), const),
            pl.BlockSpec((H, H), const),
            pl.BlockSpec((1, H), const),
            pl.BlockSpec((H, HEADW), const),
            pl.BlockSpec((1, HEADW), const),
            pl.BlockSpec((_BB, 1), lambda ib, id_: (ib, 0)),
            pl.BlockSpec((_FR, H), lambda ib, id_: (ib * _ND + id_, 0)),
        ],
        out_specs=[
            pl.BlockSpec((_FR, H), lambda ib, id_: (ib * _ND + id_, 0)),
            pl.BlockSpec((_BB, H), lambda ib, id_: (ib, 0)),
            pl.BlockSpec((_BB, NOUT), lambda ib, id_: (ib, 0)),
            pl.BlockSpec((_BB, 1), lambda ib, id_: (ib, 0)),
            pl.BlockSpec((_BB, 1), lambda ib, id_: (ib, 0)),
        ],
        out_shape=(
            jax.ShapeDtypeStruct((_ROWS, H), jnp.float32),
            jax.ShapeDtypeStruct((B, H), jnp.float32),
            jax.ShapeDtypeStruct((B, NOUT), jnp.float32),
            jax.ShapeDtypeStruct((B, 1), jnp.float32),
            jax.ShapeDtypeStruct((B, 1), jnp.int32),
        ),
        scratch_shapes=[pltpu.VMEM((_BB, H), jnp.float32)],
        compiler_params=pltpu.CompilerParams(
            dimension_semantics=("arbitrary", "arbitrary"),
        ),
    )(x, top, w1x, w1t, b1, w2, b2, wh, bh, idx2d, stack)


def _scatter_body(ns_in, p_ref, idx_ref, ns_out, sem):
    def start(r, carry):
        d = idx_ref[r] + 1
        pltpu.make_async_copy(p_ref.at[r], ns_out.at[r, d], sem).start()
        return carry

    lax.fori_loop(0, B, start, 0)

    def drain(r, carry):
        d = idx_ref[r] + 1
        pltpu.make_async_copy(p_ref.at[r], ns_out.at[r, d], sem).wait()
        return carry

    lax.fori_loop(0, B, drain, 0)


def _tc_scatter(new_stack, p, stack_idx):
    return pl.pallas_call(
        _scatter_body,
        out_shape=jax.ShapeDtypeStruct((B, DEPTH, H), jnp.float32),
        in_specs=[
            pl.BlockSpec(memory_space=pl.ANY),
            pl.BlockSpec(memory_space=pltpu.VMEM),
            pl.BlockSpec(memory_space=pltpu.SMEM),
        ],
        out_specs=pl.BlockSpec(memory_space=pl.ANY),
        input_output_aliases={0: 0},
        scratch_shapes=[pltpu.SemaphoreType.DMA],
    )(new_stack, p, stack_idx)


def kernel(x, stack, stack_idx, W1, b1, W2, b2, Ws, bs, Wp, bp, Wv, bv):
    stack_flat = stack.reshape(_ROWS, H)

    top = _sc_gather(stack_flat, stack_idx)

    w1x = W1[:OBS]
    w1t = W1[OBS:]
    wh = jnp.zeros((H, HEADW), jnp.float32)
    wh = wh.at[:, 0:3].set(Ws).at[:, 3:3 + NOUT].set(Wp)
    wh = wh.at[:, 3 + NOUT:4 + NOUT].set(Wv)
    bh = jnp.zeros((1, HEADW), jnp.float32)
    bh = bh.at[0, 0:3].set(bs).at[0, 3:3 + NOUT].set(bp)
    bh = bh.at[0, 3 + NOUT].set(bv[0])

    ns_copy, p, logits, value, nidx = _tc_mega(
        x, top, w1x, w1t, b1.reshape(1, H), W2, b2.reshape(1, H), wh, bh,
        stack_idx.reshape(B, 1), stack_flat,
    )

    new_stack = _tc_scatter(ns_copy.reshape(B, DEPTH, H), p, stack_idx)

    return (logits, value[:, 0], new_stack, nidx[:, 0])


# speedup vs baseline: 11.3363x; 1.4032x over previous
import jax
import jax.numpy as jnp
from jax import lax
from jax.experimental import pallas as pl
from jax.experimental.pallas import tpu as pltpu

B = 256
OBS = 1024
H = 2048
DEPTH = 100
NOUT = 64
HEADW = 128

_BB = 32
_NB = B // _BB
_BD = 8
_ND = (DEPTH + _BD - 1) // _BD


def _copy_body(idx_ref, stack_ref, ns_ref, top_ref):
    ns_ref[...] = stack_ref[...]
    step = pl.program_id(1)
    idxv = idx_ref[...]
    contrib = jnp.zeros((_BB, H), jnp.float32)
    for d in range(_BD):
        m_d = idxv == step * _BD + d
        contrib = contrib + jnp.where(m_d, stack_ref[:, d, :], 0.0)

    @pl.when(step == 0)
    def _():
        top_ref[...] = contrib

    @pl.when(step != 0)
    def _():
        top_ref[...] = top_ref[...] + contrib


def _tc_copy_gather(stack, idx2d):
    return pl.pallas_call(
        _copy_body,
        grid=(_NB, _ND),
        in_specs=[
            pl.BlockSpec((_BB, 1), lambda ib, id_: (ib, 0)),
            pl.BlockSpec((_BB, _BD, H), lambda ib, id_: (ib, id_, 0)),
        ],
        out_specs=[
            pl.BlockSpec((_BB, _BD, H), lambda ib, id_: (ib, id_, 0)),
            pl.BlockSpec((_BB, H), lambda ib, id_: (ib, 0)),
        ],
        out_shape=(
            jax.ShapeDtypeStruct((B, DEPTH, H), jnp.float32),
            jax.ShapeDtypeStruct((B, H), jnp.float32),
        ),
        compiler_params=pltpu.CompilerParams(
            dimension_semantics=("arbitrary", "arbitrary"),
        ),
    )(idx2d, stack)


def _mlp_body(x_ref, top_ref, w1x_ref, w1t_ref, b1_ref, w2_ref, b2_ref,
              wh_ref, bh_ref, idx_ref, p_ref, logits_ref, value_ref,
              nidx_ref):
    h = jnp.tanh(
        jnp.dot(x_ref[...], w1x_ref[...], preferred_element_type=jnp.float32)
        + jnp.dot(top_ref[...], w1t_ref[...], preferred_element_type=jnp.float32)
        + b1_ref[...]
    )
    p = jnp.tanh(
        jnp.dot(h, w2_ref[...], preferred_element_type=jnp.float32)
        + b2_ref[...]
    )
    p_ref[...] = p
    ph = jnp.dot(p, wh_ref[...], preferred_element_type=jnp.float32) + bh_ref[...]
    logits_ref[...] = ph[:, 3:3 + NOUT]
    value_ref[...] = ph[:, 3 + NOUT:4 + NOUT]
    s0 = ph[:, 0:1]
    s1 = ph[:, 1:2]
    s2 = ph[:, 2:3]
    op = jnp.where(s1 > s0, 1, 0)
    best = jnp.maximum(s0, s1)
    op = jnp.where(s2 > best, 2, op)
    nidx_ref[...] = jnp.maximum(idx_ref[...] + op - 1, 0)


def _tc_mlp(x, top, w1x, w1t, b1, w2, b2, wh, bh, idx2d):
    return pl.pallas_call(
        _mlp_body,
        out_shape=(
            jax.ShapeDtypeStruct((B, H), jnp.float32),
            jax.ShapeDtypeStruct((B, NOUT), jnp.float32),
            jax.ShapeDtypeStruct((B, 1), jnp.float32),
            jax.ShapeDtypeStruct((B, 1), jnp.int32),
        ),
    )(x, top, w1x, w1t, b1, w2, b2, wh, bh, idx2d)


def _scatter_body(ns_in, p_ref, idx_ref, ns_out, sem):
    def start(r, carry):
        d = idx_ref[r] + 1
        pltpu.make_async_copy(p_ref.at[r], ns_out.at[r, d], sem).start()
        return carry

    lax.fori_loop(0, B, start, 0)

    def drain(r, carry):
        d = idx_ref[r] + 1
        pltpu.make_async_copy(p_ref.at[r], ns_out.at[r, d], sem).wait()
        return carry

    lax.fori_loop(0, B, drain, 0)


def _tc_scatter(new_stack, p, stack_idx):
    return pl.pallas_call(
        _scatter_body,
        out_shape=jax.ShapeDtypeStruct((B, DEPTH, H), jnp.float32),
        in_specs=[
            pl.BlockSpec(memory_space=pl.ANY),
            pl.BlockSpec(memory_space=pltpu.VMEM),
            pl.BlockSpec(memory_space=pltpu.SMEM),
        ],
        out_specs=pl.BlockSpec(memory_space=pl.ANY),
        input_output_aliases={0: 0},
        scratch_shapes=[pltpu.SemaphoreType.DMA],
    )(new_stack, p, stack_idx)


def kernel(x, stack, stack_idx, W1, b1, W2, b2, Ws, bs, Wp, bp, Wv, bv):
    idx2d = stack_idx.reshape(B, 1)

    ns_copy, top = _tc_copy_gather(stack, idx2d)

    w1x = W1[:OBS]
    w1t = W1[OBS:]
    wh = jnp.zeros((H, HEADW), jnp.float32)
    wh = wh.at[:, 0:3].set(Ws).at[:, 3:3 + NOUT].set(Wp)
    wh = wh.at[:, 3 + NOUT:4 + NOUT].set(Wv)
    bh = jnp.zeros((1, HEADW), jnp.float32)
    bh = bh.at[0, 0:3].set(bs).at[0, 3:3 + NOUT].set(bp)
    bh = bh.at[0, 3 + NOUT].set(bv[0])

    p, logits, value, nidx = _tc_mlp(
        x, top, w1x, w1t, b1.reshape(1, H), W2, b2.reshape(1, H), wh, bh,
        idx2d,
    )

    new_stack = _tc_scatter(ns_copy, p, stack_idx)

    return (logits, value[:, 0], new_stack, nidx[:, 0])


# speedup vs baseline: 13.3844x; 1.1807x over previous
import jax
import jax.numpy as jnp
from jax import lax
from jax.experimental import pallas as pl
from jax.experimental.pallas import tpu as pltpu

B = 256
OBS = 1024
H = 2048
DEPTH = 100
NOUT = 64
HEADW = 128

_BB = 16
_NB = B // _BB


def _gather_body(idx_ref, stack_any, top_ref, sem):
    def start(r, carry):
        d = idx_ref[r]
        pltpu.make_async_copy(stack_any.at[r, d], top_ref.at[r], sem).start()
        return carry

    lax.fori_loop(0, B, start, 0)

    def drain(r, carry):
        d = idx_ref[r]
        pltpu.make_async_copy(stack_any.at[r, d], top_ref.at[r], sem).wait()
        return carry

    lax.fori_loop(0, B, drain, 0)


def _tc_gather(stack, stack_idx):
    return pl.pallas_call(
        _gather_body,
        out_shape=jax.ShapeDtypeStruct((B, H), jnp.float32),
        in_specs=[
            pl.BlockSpec(memory_space=pltpu.SMEM),
            pl.BlockSpec(memory_space=pl.ANY),
        ],
        out_specs=pl.BlockSpec(memory_space=pltpu.VMEM),
        scratch_shapes=[pltpu.SemaphoreType.DMA],
    )(stack_idx, stack)


def _mlp_body(x_ref, top_ref, w1_ref, b1_ref, w2_ref, b2_ref,
              wh_ref, bh_ref, idx_ref, p_ref, logits_ref, value_ref,
              nidx_ref):
    h = jnp.tanh(
        jnp.dot(x_ref[...], w1_ref[:OBS], preferred_element_type=jnp.float32)
        + jnp.dot(top_ref[...], w1_ref[OBS:],
                  preferred_element_type=jnp.float32)
        + b1_ref[...]
    )
    p = jnp.tanh(
        jnp.dot(h, w2_ref[...], preferred_element_type=jnp.float32)
        + b2_ref[...]
    )
    p_ref[...] = p
    ph = jnp.dot(p, wh_ref[...], preferred_element_type=jnp.float32) + bh_ref[...]
    logits_ref[...] = ph[:, 3:3 + NOUT]
    value_ref[...] = ph[:, 3 + NOUT:4 + NOUT]
    s0 = ph[:, 0:1]
    s1 = ph[:, 1:2]
    s2 = ph[:, 2:3]
    op = jnp.where(s1 > s0, 1, 0)
    best = jnp.maximum(s0, s1)
    op = jnp.where(s2 > best, 2, op)
    nidx_ref[...] = jnp.maximum(idx_ref[...] + op - 1, 0)


def _tc_mlp(x, top, w1, b1, w2, b2, wh, bh, idx2d):
    return pl.pallas_call(
        _mlp_body,
        out_shape=(
            jax.ShapeDtypeStruct((B, H), jnp.float32),
            jax.ShapeDtypeStruct((B, NOUT), jnp.float32),
            jax.ShapeDtypeStruct((B, 1), jnp.float32),
            jax.ShapeDtypeStruct((B, 1), jnp.int32),
        ),
    )(x, top, w1, b1, w2, b2, wh, bh, idx2d)


def _copysc_body(idx_ref, stack_ref, p_ref, ns_ref):
    ns_ref[...] = stack_ref[...]
    base = pl.program_id(0) * _BB
    for rr in range(_BB):
        d = idx_ref[base + rr] + 1
        ns_ref[rr, pl.ds(d, 1), :] = p_ref[rr:rr + 1, :]


def _tc_copy_scatter(stack, p, idx_smem):
    return pl.pallas_call(
        _copysc_body,
        grid=(_NB,),
        in_specs=[
            pl.BlockSpec(memory_space=pltpu.SMEM),
            pl.BlockSpec((_BB, DEPTH, H), lambda i: (i, 0, 0)),
            pl.BlockSpec((_BB, H), lambda i: (i, 0)),
        ],
        out_specs=pl.BlockSpec((_BB, DEPTH, H), lambda i: (i, 0, 0)),
        out_shape=jax.ShapeDtypeStruct((B, DEPTH, H), jnp.float32),
        compiler_params=pltpu.CompilerParams(
            dimension_semantics=("arbitrary",),
        ),
    )(idx_smem, stack, p)


def kernel(x, stack, stack_idx, W1, b1, W2, b2, Ws, bs, Wp, bp, Wv, bv):
    idx2d = stack_idx.reshape(B, 1)

    top = _tc_gather(stack, stack_idx)

    wh = jnp.concatenate(
        [Ws, Wp, Wv, jnp.zeros((H, HEADW - NOUT - 4), jnp.float32)], axis=1)
    bh = jnp.concatenate(
        [bs, bp, bv, jnp.zeros((HEADW - NOUT - 4,), jnp.float32)]
    ).reshape(1, HEADW)

    p, logits, value, nidx = _tc_mlp(
        x, top, W1, b1.reshape(1, H), W2, b2.reshape(1, H), wh, bh, idx2d,
    )

    new_stack = _tc_copy_scatter(stack, p, stack_idx)

    return (logits, value[:, 0], new_stack, nidx[:, 0])


# speedup vs baseline: 40.8021x; 3.0485x over previous
import jax
import jax.numpy as jnp
from jax import lax
from jax.experimental import pallas as pl
from jax.experimental.pallas import tpu as pltpu

B = 256
OBS = 1024
H = 2048
DEPTH = 100
NOUT = 64
HEADW = 128

_BD = 4
_ND = DEPTH // _BD


def _gather_body(idx_ref, stackt_any, top_ref, sem):
    def start(r, carry):
        d = idx_ref[r]
        pltpu.make_async_copy(stackt_any.at[d, r], top_ref.at[r], sem).start()
        return carry

    lax.fori_loop(0, B, start, 0)

    def drain(r, carry):
        d = idx_ref[r]
        pltpu.make_async_copy(stackt_any.at[d, r], top_ref.at[r], sem).wait()
        return carry

    lax.fori_loop(0, B, drain, 0)


def _tc_gather(stack_t, stack_idx):
    return pl.pallas_call(
        _gather_body,
        out_shape=jax.ShapeDtypeStruct((B, H), jnp.float32),
        in_specs=[
            pl.BlockSpec(memory_space=pltpu.SMEM),
            pl.BlockSpec(memory_space=pl.ANY),
        ],
        out_specs=pl.BlockSpec(memory_space=pltpu.VMEM),
        scratch_shapes=[pltpu.SemaphoreType.DMA],
    )(stack_idx, stack_t)


def _mlp_body(x_ref, top_ref, w1_ref, b1_ref, w2_ref, b2_ref,
              wh_ref, bh_ref, idx_ref, p_ref, logits_ref, value_ref,
              nidx_ref):
    h = jnp.tanh(
        jnp.dot(x_ref[...], w1_ref[:OBS], preferred_element_type=jnp.float32)
        + jnp.dot(top_ref[...], w1_ref[OBS:],
                  preferred_element_type=jnp.float32)
        + b1_ref[...]
    )
    p = jnp.tanh(
        jnp.dot(h, w2_ref[...], preferred_element_type=jnp.float32)
        + b2_ref[...]
    )
    p_ref[...] = p
    ph = jnp.dot(p, wh_ref[...], preferred_element_type=jnp.float32) + bh_ref[...]
    logits_ref[...] = ph[:, 3:3 + NOUT]
    value_ref[...] = ph[:, 3 + NOUT:4 + NOUT]
    s0 = ph[:, 0:1]
    s1 = ph[:, 1:2]
    s2 = ph[:, 2:3]
    op = jnp.where(s1 > s0, 1, 0)
    best = jnp.maximum(s0, s1)
    op = jnp.where(s2 > best, 2, op)
    nidx_ref[...] = jnp.maximum(idx_ref[...] + op - 1, 0)


def _tc_mlp(x, top, w1, b1, w2, b2, wh, bh, idx2d):
    return pl.pallas_call(
        _mlp_body,
        out_shape=(
            jax.ShapeDtypeStruct((B, H), jnp.float32),
            jax.ShapeDtypeStruct((B, NOUT), jnp.float32),
            jax.ShapeDtypeStruct((B, 1), jnp.float32),
            jax.ShapeDtypeStruct((B, 1), jnp.int32),
        ),
    )(x, top, w1, b1, w2, b2, wh, bh, idx2d)


def _copysc_body(idx_ref, stack_ref, p_ref, ns_ref):
    base = pl.program_id(0) * _BD
    tgt = idx_ref[...] + 1
    for d in range(_BD):
        m = tgt == base + d
        ns_ref[d] = jnp.where(m, p_ref[...], stack_ref[d])


def _tc_copy_scatter(stack_t, p, idx2d):
    return pl.pallas_call(
        _copysc_body,
        grid=(_ND,),
        in_specs=[
            pl.BlockSpec((B, 1), lambda i: (0, 0)),
            pl.BlockSpec((_BD, B, H), lambda i: (i, 0, 0)),
            pl.BlockSpec((B, H), lambda i: (0, 0)),
        ],
        out_specs=pl.BlockSpec((_BD, B, H), lambda i: (i, 0, 0)),
        out_shape=jax.ShapeDtypeStruct((DEPTH, B, H), jnp.float32),
        compiler_params=pltpu.CompilerParams(
            dimension_semantics=("arbitrary",),
        ),
    )(idx2d, stack_t, p)


def kernel(x, stack, stack_idx, W1, b1, W2, b2, Ws, bs, Wp, bp, Wv, bv):
    idx2d = stack_idx.reshape(B, 1)
    stack_t = jnp.swapaxes(stack, 0, 1)

    top = _tc_gather(stack_t, stack_idx)

    wh = jnp.concatenate(
        [Ws, Wp, Wv, jnp.zeros((H, HEADW - NOUT - 4), jnp.float32)], axis=1)
    bh = jnp.concatenate(
        [bs, bp, bv, jnp.zeros((HEADW - NOUT - 4,), jnp.float32)]
    ).reshape(1, HEADW)

    p, logits, value, nidx = _tc_mlp(
        x, top, W1, b1.reshape(1, H), W2, b2.reshape(1, H), wh, bh, idx2d,
    )

    new_stack_t = _tc_copy_scatter(stack_t, p, idx2d)
    new_stack = jnp.swapaxes(new_stack_t, 0, 1)

    return (logits, value[:, 0], new_stack, nidx[:, 0])
